# baseline (speedup 1.0000x reference)
import functools

import jax
import jax.numpy as jnp
from jax import lax
from jax.experimental import pallas as pl
from jax.experimental.pallas import tpu as pltpu
from jax.experimental.pallas import tpu_sc as plsc

_N = 4096
_MAXP = 64 * _N
_OUT = _MAXP + 128
_NC = 2
_NS = 16
_NW = _NC * _NS
_RPW = _N // _NW
_SLICE = _MAXP // _NW
_CUT2 = 0.2 * 0.2
_FLUSH_T = 8192
_STAGE = 12352

_mesh = plsc.VectorSubcoreMesh(core_axis_name="c", subcore_axis_name="s")


def _it16():
    return lax.iota(jnp.int32, 16)


def _splat(i):
    return jnp.zeros((16,), jnp.int32) + i


def _seg_bounds(bt_v):
    b = jnp.minimum(_it16(), 8)
    lo = jnp.zeros((16,), jnp.int32)
    hi = jnp.full((16,), _N, jnp.int32)
    for _ in range(12):
        mid = (lo + hi) >> 1
        v = plsc.load_gather(bt_v, [mid])
        p = v < b
        lo = jnp.where(p, mid + 1, lo)
        hi = jnp.where(p, hi, mid)
    return lo


def _row_ctx(i, xs_v, ys_v, zs_v, bt_v, bnd_v):
    iv = _splat(i)
    xi = plsc.load_gather(xs_v, [iv])
    yi = plsc.load_gather(ys_v, [iv])
    zi = plsc.load_gather(zs_v, [iv])
    bi = plsc.load_gather(bt_v, [iv])
    s = jnp.max(plsc.load_gather(bnd_v, [bi]))
    e = jnp.max(plsc.load_gather(bnd_v, [bi + 1]))
    return iv, xi, yi, zi, s, e


def _chunk_mask(base, it, iv, xi, yi, zi, s, e, xs_v, ys_v, zs_v):
    jv = base + it
    dx = xs_v[pl.ds(base, 16)] - xi
    dy = ys_v[pl.ds(base, 16)] - yi
    dz = zs_v[pl.ds(base, 16)] - zi
    d2 = dx * dx + dy * dy + dz * dz
    m = (jv >= s) & (jv < e) & (jv != iv) & (d2 < _CUT2)
    return jv, d2, m


def _sqrt16(d2):
    y = plsc.bitcast(0x5F3759DF - (plsc.bitcast(d2, jnp.int32) >> 1),
                     jnp.float32)
    for _ in range(3):
        y = y * (1.5 - 0.5 * d2 * y * y)
    return jnp.where(d2 > 0, d2 * y, 0.0)


def _count_body(xs_h, ys_h, zs_h, bt_h, tot_h, xs_v, ys_v, zs_v, bt_v, bnd_v,
                tot_v, csem):
    wid = lax.axis_index("s") * _NC + lax.axis_index("c")
    d0 = pltpu.async_copy(xs_h, xs_v.at[pl.ds(0, _N)], csem)
    d1 = pltpu.async_copy(ys_h, ys_v.at[pl.ds(0, _N)], csem)
    d2 = pltpu.async_copy(zs_h, zs_v.at[pl.ds(0, _N)], csem)
    d3 = pltpu.async_copy(bt_h, bt_v, csem)
    d0.wait()
    d1.wait()
    d2.wait()
    d3.wait()
    bnd_v[...] = _seg_bounds(bt_v)
    it = _it16()
    row0 = wid * _RPW

    def row_body(i, acc):
        iv, xi, yi, zi, s, e = _row_ctx(i, xs_v, ys_v, zs_v, bt_v, bnd_v)
        c0 = s >> 4
        nch = ((e + 15) >> 4) - c0

        def chunk(k, a):
            base = (c0 + 4 * k) * 16
            for u in range(4):
                _, _, m = _chunk_mask(base + u * 16, it, iv, xi, yi, zi, s, e,
                                      xs_v, ys_v, zs_v)
                a = a + plsc.all_reduce_population_count(m)
            return a

        return lax.fori_loop(0, (nch + 3) >> 2, chunk, acc)

    tot = lax.fori_loop(row0, row0 + _RPW, row_body, jnp.zeros((16,), jnp.int32))
    tot_v[...] = jnp.zeros((16,), jnp.int32) + jnp.max(tot)
    pltpu.sync_copy(tot_v, tot_h.at[pl.ds(wid * 16, 16)])


_params = pltpu.CompilerParams(needs_layout_passes=False)

_count_call = pl.kernel(
    _count_body,
    out_type=jax.ShapeDtypeStruct((_NW * 16,), jnp.int32),
    mesh=_mesh,
    compiler_params=_params,
    scratch_types=[
        pltpu.VMEM((_N + 64,), jnp.float32),
        pltpu.VMEM((_N + 64,), jnp.float32),
        pltpu.VMEM((_N + 64,), jnp.float32),
        pltpu.VMEM((_N,), jnp.int32),
        pltpu.VMEM((16,), jnp.int32),
        pltpu.VMEM((16,), jnp.int32),
        pltpu.SemaphoreType.DMA,
    ],
)


def _emit_body(xs_h, ys_h, zs_h, bt_h, tot_h, ei0_h, ei1_h, ew_h,
               xs_v, ys_v, zs_v, bt_v, bnd_v, tot_v,
               stji, std, vi_v, vj_v, vf_v,
               idx16, vih, vjh, vfh, cneg_v, czer_v,
               sem0, sem1, sem2):
    wid = lax.axis_index("s") * _NC + lax.axis_index("c")
    d0 = pltpu.async_copy(xs_h, xs_v.at[pl.ds(0, _N)], sem0)
    d1 = pltpu.async_copy(ys_h, ys_v.at[pl.ds(0, _N)], sem1)
    d2 = pltpu.async_copy(zs_h, zs_v.at[pl.ds(0, _N)], sem2)
    d3 = pltpu.async_copy(bt_h, bt_v, sem0)
    d4 = pltpu.async_copy(tot_h, tot_v, sem1)
    d0.wait()
    d1.wait()
    d2.wait()
    d3.wait()
    d4.wait()
    bnd_v[...] = _seg_bounds(bt_v)
    it = _it16()

    def tot_body(c, carry):
        ba, ta = carry
        ch = tot_v[pl.ds(c * 16, 16)]
        lane0 = it == 0
        v0 = jnp.where(lane0, ch, 0)
        return (ba + jnp.where(c < wid, v0, 0), ta + v0)

    ba, ta = lax.fori_loop(0, _NW, tot_body,
                           (jnp.zeros((16,), jnp.int32),
                            jnp.zeros((16,), jnp.int32)))
    base = jnp.sum(ba)
    total = jnp.minimum(jnp.sum(ta), _MAXP)

    def const_body(k, _):
        sl = pl.ds(k * 16, 16)
        cneg_v[sl] = jnp.full((16,), -1, jnp.int32)
        czer_v[sl] = jnp.zeros((16,), jnp.float32)
        return 0

    lax.fori_loop(0, 128, const_body, 0)

    def scat16(pos0, cnt, soff):
        gi = soff + it
        ji = plsc.load_gather(stji, [gi])
        vjh[...] = ji & 4095
        vih[...] = lax.shift_right_logical(ji, 12)
        vfh[...] = _sqrt16(plsc.load_gather(std, [gi]))
        idx16[...] = jnp.where(it < cnt, pos0 + it, _MAXP)
        d0 = pltpu.async_copy(vih, ei0_h.at[idx16], sem0)
        d1 = pltpu.async_copy(vjh, ei1_h.at[idx16], sem1)
        d2_ = pltpu.async_copy(vfh, ew_h.at[idx16], sem2)
        d0.wait()
        d1.wait()
        d2_.wait()

    pad_lo = jnp.maximum(wid * _SLICE, total)
    pad_hi = (wid + 1) * _SLICE
    npad = jnp.maximum(pad_hi - pad_lo, 0)
    phead = jnp.minimum((8 - pad_lo % 8) % 8, npad)

    @pl.when(phead > 0)
    def _():
        idx16[...] = jnp.where(it < phead, pad_lo + it, _MAXP)
        vfh[...] = jnp.zeros((16,), jnp.float32)
        d0 = pltpu.async_copy(cneg_v.at[pl.ds(0, 16)], ei0_h.at[idx16], sem0)
        d1 = pltpu.async_copy(cneg_v.at[pl.ds(0, 16)], ei1_h.at[idx16], sem1)
        d2_ = pltpu.async_copy(vfh, ew_h.at[idx16], sem2)
        d0.wait()
        d1.wait()
        d2_.wait()

    pal = pl.multiple_of(pad_lo + phead, 8)
    plen = jnp.maximum(pad_hi - pal, 0)

    def pad_chunk(k, _):
        cur_ = pl.multiple_of(pal + k * 2048, 8)
        d0 = pltpu.async_copy(cneg_v, ei0_h.at[pl.ds(cur_, 2048)], sem0)
        d1 = pltpu.async_copy(cneg_v, ei1_h.at[pl.ds(cur_, 2048)], sem1)
        d2 = pltpu.async_copy(czer_v, ew_h.at[pl.ds(cur_, 2048)], sem2)
        d0.wait()
        d1.wait()
        d2.wait()
        return 0

    lax.fori_loop(0, plen >> 11, pad_chunk, 0)
    pcur = pal + ((plen >> 11) << 11)
    for sz in (1024, 512, 256, 128, 64, 32, 16, 8):
        pred = (plen & sz) != 0

        @pl.when(pred)
        def _(pcur=pl.multiple_of(pcur, 8), sz=sz):
            d0 = pltpu.async_copy(cneg_v.at[pl.ds(0, sz)],
                                  ei0_h.at[pl.ds(pcur, sz)], sem0)
            d1 = pltpu.async_copy(cneg_v.at[pl.ds(0, sz)],
                                  ei1_h.at[pl.ds(pcur, sz)], sem1)
            d2 = pltpu.async_copy(czer_v.at[pl.ds(0, sz)],
                                  ew_h.at[pl.ds(pcur, sz)], sem2)
            d0.wait()
            d1.wait()
            d2.wait()

        pcur = pcur + jnp.where(pred, sz, 0)

    def flush(n_f, fb):
        a = jnp.clip(_MAXP - fb, 0, n_f)
        head = jnp.minimum((8 - fb % 8) % 8, a)

        @pl.when(head > 0)
        def _():
            scat16(fb, head, 0)

        lf = (a - head) & -8

        def copy_body(k, _):
            gi = head + k * 16 + it
            dst = pl.ds(k * 16, 16)
            ji = plsc.load_gather(stji, [gi])
            vj_v[dst] = ji & 4095
            vi_v[dst] = lax.shift_right_logical(ji, 12)
            vf_v[dst] = _sqrt16(plsc.load_gather(std, [gi]))
            return 0

        lax.fori_loop(0, (lf + 15) >> 4, copy_body, 0)

        cur = fb + head
        soff = jnp.int32(0)
        for sz in (8192, 4096, 2048, 1024, 512, 256, 128, 64, 32, 16, 8):
            pred = (lf & sz) != 0

            @pl.when(pred)
            def _(cur=pl.multiple_of(cur, 8), soff=pl.multiple_of(soff, 8),
                  sz=sz):
                d0 = pltpu.async_copy(vi_v.at[pl.ds(soff, sz)],
                                      ei0_h.at[pl.ds(cur, sz)], sem0)
                d1 = pltpu.async_copy(vj_v.at[pl.ds(soff, sz)],
                                      ei1_h.at[pl.ds(cur, sz)], sem1)
                d2 = pltpu.async_copy(vf_v.at[pl.ds(soff, sz)],
                                      ew_h.at[pl.ds(cur, sz)], sem2)
                d0.wait()
                d1.wait()
                d2.wait()

            inc = jnp.where(pred, sz, 0)
            cur = cur + inc
            soff = soff + inc

        rem = a - head - lf

        @pl.when(rem > 0)
        def _(cur=cur):
            scat16(cur, rem, head + lf)

    row0 = wid * _RPW

    def row_body(i, carry):
        wp_v, fb = carry
        iv, xi, yi, zi, s, e = _row_ctx(i, xs_v, ys_v, zs_v, bt_v, bnd_v)
        iv12 = iv * 4096
        c0 = s >> 4
        nch = ((e + 15) >> 4) - c0

        def chunk(k, wpv):
            base = (c0 + 4 * k) * 16
            jvs, d2s, ms = [], [], []
            for u in range(4):
                jv, d2, m = _chunk_mask(base + u * 16, it, iv, xi, yi, zi,
                                        s, e, xs_v, ys_v, zs_v)
                jvs.append(jv)
                d2s.append(d2)
                ms.append(m)

            css = [plsc.cumsum(jnp.where(m, 1, 0)) for m in ms]
            pcs = [plsc.all_reduce_population_count(m) for m in ms]
            for u in range(4):
                pos = wpv + css[u] - 1
                plsc.store_scatter(stji, [pos], iv12 + jvs[u], mask=ms[u])
                plsc.store_scatter(std, [pos], d2s[u], mask=ms[u])
                wpv = wpv + pcs[u]
            return wpv

        wp_v = lax.fori_loop(0, (nch + 3) >> 2, chunk, wp_v)
        wp_s = jnp.max(wp_v)

        def do_flush(args):
            wpv_, fb_ = args
            flush(_FLUSH_T, fb_)
            nshift = jnp.max(wpv_) - _FLUSH_T

            def shift_body(k, _):
                src = pl.ds(_FLUSH_T + k * 16, 16)
                dst = pl.ds(k * 16, 16)
                stji[dst] = stji[src]
                std[dst] = std[src]
                return 0

            lax.fori_loop(0, (nshift + 15) >> 4, shift_body, 0)
            return wpv_ - _FLUSH_T, fb_ + _FLUSH_T

        return lax.cond(wp_s >= _FLUSH_T, do_flush, lambda a_: a_, (wp_v, fb))

    wp_v, fb = lax.fori_loop(row0, row0 + _RPW, row_body,
                             (jnp.zeros((16,), jnp.int32), base))
    flush(jnp.max(wp_v), fb)


_emit_call = pl.kernel(
    _emit_body,
    out_type=(
        jax.ShapeDtypeStruct((_OUT,), jnp.int32),
        jax.ShapeDtypeStruct((_OUT,), jnp.int32),
        jax.ShapeDtypeStruct((_OUT,), jnp.float32),
    ),
    mesh=_mesh,
    compiler_params=_params,
    scratch_types=[
        pltpu.VMEM((_N + 64,), jnp.float32),
        pltpu.VMEM((_N + 64,), jnp.float32),
        pltpu.VMEM((_N + 64,), jnp.float32),
        pltpu.VMEM((_N,), jnp.int32),
        pltpu.VMEM((16,), jnp.int32),
        pltpu.VMEM((_NW * 16,), jnp.int32),
        pltpu.VMEM((_STAGE,), jnp.int32),
        pltpu.VMEM((_STAGE,), jnp.float32),
        pltpu.VMEM((_STAGE,), jnp.int32),
        pltpu.VMEM((_STAGE,), jnp.int32),
        pltpu.VMEM((_STAGE,), jnp.float32),
        pltpu.VMEM((16,), jnp.int32),
        pltpu.VMEM((16,), jnp.int32),
        pltpu.VMEM((16,), jnp.int32),
        pltpu.VMEM((16,), jnp.float32),
        pltpu.VMEM((2048,), jnp.int32),
        pltpu.VMEM((2048,), jnp.float32),
        pltpu.SemaphoreType.DMA,
        pltpu.SemaphoreType.DMA,
        pltpu.SemaphoreType.DMA,
    ],
)


def kernel(pos, batch):
    pt = pos.astype(jnp.float32).T
    xs, ys, zs = pt[0], pt[1], pt[2]
    bt = batch.astype(jnp.int32)
    tot = _count_call(xs, ys, zs, bt)
    ei0, ei1, ew = _emit_call(xs, ys, zs, bt, tot)
    edge_index = jnp.stack([ei0[:_MAXP], ei1[:_MAXP]])
    edge_weight = ew[:_MAXP]
    return edge_index, edge_weight, None


# speedup vs baseline: 1.0002x; 1.0002x over previous
import jax
import jax.numpy as jnp
from jax import lax
from jax.experimental import pallas as pl
from jax.experimental.pallas import tpu as pltpu
from jax.experimental.pallas import tpu_sc as plsc

_N = 4096
_MAXP = 64 * _N
_OUT = _MAXP + 128
_NC = 2
_NS = 16
_NW = _NC * _NS
_RPW = _N // _NW
_SLICE = _MAXP // _NW
_CUT2 = 0.2 * 0.2
_FLUSH_T = 8192
_STAGE = 12352

_mesh = plsc.VectorSubcoreMesh(core_axis_name="c", subcore_axis_name="s")


def _it16():
    return lax.iota(jnp.int32, 16)


def _splat(i):
    return jnp.zeros((16,), jnp.int32) + i


def _seg_bounds(bt_v):
    b = jnp.minimum(_it16(), 8)
    lo = jnp.zeros((16,), jnp.int32)
    hi = jnp.full((16,), _N, jnp.int32)
    for _ in range(12):
        mid = (lo + hi) >> 1
        v = plsc.load_gather(bt_v, [mid])
        p = v < b
        lo = jnp.where(p, mid + 1, lo)
        hi = jnp.where(p, hi, mid)
    return lo


def _row_ctx(i, xs_v, ys_v, zs_v, bt_v, bnd_v):
    iv = _splat(i)
    xi = plsc.load_gather(xs_v, [iv])
    yi = plsc.load_gather(ys_v, [iv])
    zi = plsc.load_gather(zs_v, [iv])
    bi = plsc.load_gather(bt_v, [iv])
    s = jnp.max(plsc.load_gather(bnd_v, [bi]))
    e = jnp.max(plsc.load_gather(bnd_v, [bi + 1]))
    return iv, xi, yi, zi, s, e


def _chunk_mask(base, it, iv, xi, yi, zi, s, e, xs_v, ys_v, zs_v):
    jv = base + it
    dx = xs_v[pl.ds(base, 16)] - xi
    dy = ys_v[pl.ds(base, 16)] - yi
    dz = zs_v[pl.ds(base, 16)] - zi
    d2 = dx * dx + dy * dy + dz * dz
    m = (jv >= s) & (jv < e) & (jv != iv) & (d2 < _CUT2)
    return jv, d2, m


def _sqrt16(d2):
    y = plsc.bitcast(0x5F3759DF - (plsc.bitcast(d2, jnp.int32) >> 1),
                     jnp.float32)
    for _ in range(3):
        y = y * (1.5 - 0.5 * d2 * y * y)
    return jnp.where(d2 > 0, d2 * y, 0.0)


def _count_body(xs_h, ys_h, zs_h, bt_h, tot_h, xs_v, ys_v, zs_v, bt_v, bnd_v,
                tot_v, csem):
    wid = lax.axis_index("s") * _NC + lax.axis_index("c")
    d0 = pltpu.async_copy(xs_h, xs_v.at[pl.ds(0, _N)], csem)
    d1 = pltpu.async_copy(ys_h, ys_v.at[pl.ds(0, _N)], csem)
    d2 = pltpu.async_copy(zs_h, zs_v.at[pl.ds(0, _N)], csem)
    d3 = pltpu.async_copy(bt_h, bt_v, csem)
    d0.wait()
    d1.wait()
    d2.wait()
    d3.wait()
    bnd_v[...] = _seg_bounds(bt_v)
    it = _it16()
    row0 = wid * _RPW

    def row_body(i, acc):
        iv, xi, yi, zi, s, e = _row_ctx(i, xs_v, ys_v, zs_v, bt_v, bnd_v)
        c0 = s >> 4
        nch = ((e + 15) >> 4) - c0

        def chunk(k, a):
            base = (c0 + 4 * k) * 16
            for u in range(4):
                _, _, m = _chunk_mask(base + u * 16, it, iv, xi, yi, zi, s, e,
                                      xs_v, ys_v, zs_v)
                a = a + plsc.all_reduce_population_count(m)
            return a

        return lax.fori_loop(0, (nch + 3) >> 2, chunk, acc)

    tot = lax.fori_loop(row0, row0 + _RPW, row_body, jnp.zeros((16,), jnp.int32))
    tot_v[...] = jnp.zeros((16,), jnp.int32) + jnp.max(tot)
    pltpu.sync_copy(tot_v, tot_h.at[pl.ds(wid * 16, 16)])


_params = pltpu.CompilerParams(needs_layout_passes=False)

_count_call = pl.kernel(
    _count_body,
    out_type=jax.ShapeDtypeStruct((_NW * 16,), jnp.int32),
    mesh=_mesh,
    compiler_params=_params,
    scratch_types=[
        pltpu.VMEM((_N + 64,), jnp.float32),
        pltpu.VMEM((_N + 64,), jnp.float32),
        pltpu.VMEM((_N + 64,), jnp.float32),
        pltpu.VMEM((_N,), jnp.int32),
        pltpu.VMEM((16,), jnp.int32),
        pltpu.VMEM((16,), jnp.int32),
        pltpu.SemaphoreType.DMA,
    ],
)


def _emit_body(xs_h, ys_h, zs_h, bt_h, tot_h, ei0_h, ei1_h, ew_h,
               xs_v, ys_v, zs_v, bt_v, bnd_v, tot_v,
               stji, std, vi_v, vj_v, vf_v,
               idx16, vih, vjh, vfh, cneg_v, czer_v,
               sem0, sem1, sem2):
    wid = lax.axis_index("s") * _NC + lax.axis_index("c")
    d0 = pltpu.async_copy(xs_h, xs_v.at[pl.ds(0, _N)], sem0)
    d1 = pltpu.async_copy(ys_h, ys_v.at[pl.ds(0, _N)], sem1)
    d2 = pltpu.async_copy(zs_h, zs_v.at[pl.ds(0, _N)], sem2)
    d3 = pltpu.async_copy(bt_h, bt_v, sem0)
    d4 = pltpu.async_copy(tot_h, tot_v, sem1)
    d0.wait()
    d1.wait()
    d2.wait()
    d3.wait()
    d4.wait()
    bnd_v[...] = _seg_bounds(bt_v)
    it = _it16()

    def tot_body(c, carry):
        ba, ta = carry
        ch = tot_v[pl.ds(c * 16, 16)]
        lane0 = it == 0
        v0 = jnp.where(lane0, ch, 0)
        return (ba + jnp.where(c < wid, v0, 0), ta + v0)

    ba, ta = lax.fori_loop(0, _NW, tot_body,
                           (jnp.zeros((16,), jnp.int32),
                            jnp.zeros((16,), jnp.int32)))
    base = jnp.sum(ba)
    total = jnp.minimum(jnp.sum(ta), _MAXP)

    def const_body(k, _):
        sl = pl.ds(k * 16, 16)
        cneg_v[sl] = jnp.full((16,), -1, jnp.int32)
        czer_v[sl] = jnp.zeros((16,), jnp.float32)
        return 0

    lax.fori_loop(0, 128, const_body, 0)

    def scat16(pos0, cnt, soff):
        gi = soff + it
        ji = plsc.load_gather(stji, [gi])
        vjh[...] = ji & 4095
        vih[...] = lax.shift_right_logical(ji, 12)
        vfh[...] = _sqrt16(plsc.load_gather(std, [gi]))
        idx16[...] = jnp.where(it < cnt, pos0 + it, _MAXP)
        d0 = pltpu.async_copy(vih, ei0_h.at[idx16], sem0)
        d1 = pltpu.async_copy(vjh, ei1_h.at[idx16], sem1)
        d2_ = pltpu.async_copy(vfh, ew_h.at[idx16], sem2)
        d0.wait()
        d1.wait()
        d2_.wait()

    pad_lo = jnp.maximum(wid * _SLICE, total)
    pad_hi = (wid + 1) * _SLICE
    npad = jnp.maximum(pad_hi - pad_lo, 0)
    phead = jnp.minimum((8 - pad_lo % 8) % 8, npad)

    @pl.when(phead > 0)
    def _():
        idx16[...] = jnp.where(it < phead, pad_lo + it, _MAXP)
        vfh[...] = jnp.zeros((16,), jnp.float32)
        d0 = pltpu.async_copy(cneg_v.at[pl.ds(0, 16)], ei0_h.at[idx16], sem0)
        d1 = pltpu.async_copy(cneg_v.at[pl.ds(0, 16)], ei1_h.at[idx16], sem1)
        d2_ = pltpu.async_copy(vfh, ew_h.at[idx16], sem2)
        d0.wait()
        d1.wait()
        d2_.wait()

    pal = pl.multiple_of(pad_lo + phead, 8)
    plen = jnp.maximum(pad_hi - pal, 0)

    def pad_chunk(k, _):
        cur_ = pl.multiple_of(pal + k * 2048, 8)
        d0 = pltpu.async_copy(cneg_v, ei0_h.at[pl.ds(cur_, 2048)], sem0)
        d1 = pltpu.async_copy(cneg_v, ei1_h.at[pl.ds(cur_, 2048)], sem1)
        d2 = pltpu.async_copy(czer_v, ew_h.at[pl.ds(cur_, 2048)], sem2)
        d0.wait()
        d1.wait()
        d2.wait()
        return 0

    lax.fori_loop(0, plen >> 11, pad_chunk, 0)
    pcur = pal + ((plen >> 11) << 11)
    for sz in (1024, 512, 256, 128, 64, 32, 16, 8):
        pred = (plen & sz) != 0

        @pl.when(pred)
        def _(pcur=pl.multiple_of(pcur, 8), sz=sz):
            d0 = pltpu.async_copy(cneg_v.at[pl.ds(0, sz)],
                                  ei0_h.at[pl.ds(pcur, sz)], sem0)
            d1 = pltpu.async_copy(cneg_v.at[pl.ds(0, sz)],
                                  ei1_h.at[pl.ds(pcur, sz)], sem1)
            d2 = pltpu.async_copy(czer_v.at[pl.ds(0, sz)],
                                  ew_h.at[pl.ds(pcur, sz)], sem2)
            d0.wait()
            d1.wait()
            d2.wait()

        pcur = pcur + jnp.where(pred, sz, 0)

    def flush(n_f, fb):
        a = jnp.clip(_MAXP - fb, 0, n_f)
        head = jnp.minimum((8 - fb % 8) % 8, a)

        @pl.when(head > 0)
        def _():
            scat16(fb, head, 0)

        lf = (a - head) & -8

        def copy_body(k, _):
            gi = head + k * 16 + it
            dst = pl.ds(k * 16, 16)
            ji = plsc.load_gather(stji, [gi])
            vj_v[dst] = ji & 4095
            vi_v[dst] = lax.shift_right_logical(ji, 12)
            vf_v[dst] = _sqrt16(plsc.load_gather(std, [gi]))
            return 0

        lax.fori_loop(0, (lf + 15) >> 4, copy_body, 0)

        cur = fb + head
        soff = jnp.int32(0)
        for sz in (8192, 4096, 2048, 1024, 512, 256, 128, 64, 32, 16, 8):
            pred = (lf & sz) != 0

            @pl.when(pred)
            def _(cur=pl.multiple_of(cur, 8), soff=pl.multiple_of(soff, 8),
                  sz=sz):
                d0 = pltpu.async_copy(vi_v.at[pl.ds(soff, sz)],
                                      ei0_h.at[pl.ds(cur, sz)], sem0)
                d1 = pltpu.async_copy(vj_v.at[pl.ds(soff, sz)],
                                      ei1_h.at[pl.ds(cur, sz)], sem1)
                d2 = pltpu.async_copy(vf_v.at[pl.ds(soff, sz)],
                                      ew_h.at[pl.ds(cur, sz)], sem2)
                d0.wait()
                d1.wait()
                d2.wait()

            inc = jnp.where(pred, sz, 0)
            cur = cur + inc
            soff = soff + inc

        rem = a - head - lf

        @pl.when(rem > 0)
        def _(cur=cur):
            scat16(cur, rem, head + lf)

    row0 = wid * _RPW

    def row_body(i, carry):
        wp_v, fb = carry
        iv, xi, yi, zi, s, e = _row_ctx(i, xs_v, ys_v, zs_v, bt_v, bnd_v)
        iv12 = iv * 4096
        c0 = s >> 4
        nch = ((e + 15) >> 4) - c0

        def chunk(k, wpv):
            base = (c0 + 4 * k) * 16
            jvs, d2s, ms = [], [], []
            for u in range(4):
                jv, d2, m = _chunk_mask(base + u * 16, it, iv, xi, yi, zi,
                                        s, e, xs_v, ys_v, zs_v)
                jvs.append(jv)
                d2s.append(d2)
                ms.append(m)

            css = [plsc.cumsum(jnp.where(m, 1, 0)) for m in ms]
            pcs = [plsc.all_reduce_population_count(m) for m in ms]
            for u in range(4):
                pos = wpv + css[u] - 1
                plsc.store_scatter(stji, [pos], iv12 + jvs[u], mask=ms[u])
                plsc.store_scatter(std, [pos], d2s[u], mask=ms[u])
                wpv = wpv + pcs[u]
            return wpv

        wp_v = lax.fori_loop(0, (nch + 3) >> 2, chunk, wp_v)
        wp_s = jnp.max(wp_v)

        def do_flush(args):
            wpv_, fb_ = args
            flush(_FLUSH_T, fb_)
            nshift = jnp.max(wpv_) - _FLUSH_T

            def shift_body(k, _):
                src = pl.ds(_FLUSH_T + k * 16, 16)
                dst = pl.ds(k * 16, 16)
                stji[dst] = stji[src]
                std[dst] = std[src]
                return 0

            lax.fori_loop(0, (nshift + 15) >> 4, shift_body, 0)
            return wpv_ - _FLUSH_T, fb_ + _FLUSH_T

        return lax.cond(wp_s >= _FLUSH_T, do_flush, lambda a_: a_, (wp_v, fb))

    wp_v, fb = lax.fori_loop(row0, row0 + _RPW, row_body,
                             (jnp.zeros((16,), jnp.int32), base))
    flush(jnp.max(wp_v), fb)


_emit_call = pl.kernel(
    _emit_body,
    out_type=(
        jax.ShapeDtypeStruct((_OUT,), jnp.int32),
        jax.ShapeDtypeStruct((_OUT,), jnp.int32),
        jax.ShapeDtypeStruct((_OUT,), jnp.float32),
    ),
    mesh=_mesh,
    compiler_params=_params,
    scratch_types=[
        pltpu.VMEM((_N + 64,), jnp.float32),
        pltpu.VMEM((_N + 64,), jnp.float32),
        pltpu.VMEM((_N + 64,), jnp.float32),
        pltpu.VMEM((_N,), jnp.int32),
        pltpu.VMEM((16,), jnp.int32),
        pltpu.VMEM((_NW * 16,), jnp.int32),
        pltpu.VMEM((_STAGE,), jnp.int32),
        pltpu.VMEM((_STAGE,), jnp.float32),
        pltpu.VMEM((_STAGE,), jnp.int32),
        pltpu.VMEM((_STAGE,), jnp.int32),
        pltpu.VMEM((_STAGE,), jnp.float32),
        pltpu.VMEM((16,), jnp.int32),
        pltpu.VMEM((16,), jnp.int32),
        pltpu.VMEM((16,), jnp.int32),
        pltpu.VMEM((16,), jnp.float32),
        pltpu.VMEM((2048,), jnp.int32),
        pltpu.VMEM((2048,), jnp.float32),
        pltpu.SemaphoreType.DMA,
        pltpu.SemaphoreType.DMA,
        pltpu.SemaphoreType.DMA,
    ],
)


def kernel(pos, batch):
    pt = pos.astype(jnp.float32).T
    xs, ys, zs = pt[0], pt[1], pt[2]
    bt = batch.astype(jnp.int32)
    tot = _count_call(xs, ys, zs, bt)
    ei0, ei1, ew = _emit_call(xs, ys, zs, bt, tot)
    edge_index = jnp.stack([ei0[:_MAXP], ei1[:_MAXP]])
    edge_weight = ew[:_MAXP]
    return edge_index, edge_weight, None


# speedup vs baseline: 1.0160x; 1.0157x over previous
import jax
import jax.numpy as jnp
from jax import lax
from jax.experimental import pallas as pl
from jax.experimental.pallas import tpu as pltpu
from jax.experimental.pallas import tpu_sc as plsc

_N = 4096
_MAXP = 64 * _N
_OUT = _MAXP + 128
_NC = 2
_NS = 16
_NW = _NC * _NS
_RPW = _N // _NW
_SLICE = _MAXP // _NW
_CUT2 = 0.2 * 0.2
_FLUSH_T = 8192
_STAGE = 12352

_mesh = plsc.VectorSubcoreMesh(core_axis_name="c", subcore_axis_name="s")


def _it16():
    return lax.iota(jnp.int32, 16)


def _splat(i):
    return jnp.zeros((16,), jnp.int32) + i


def _seg_bounds(bt_v):
    b = jnp.minimum(_it16(), 8)
    lo = jnp.zeros((16,), jnp.int32)
    hi = jnp.full((16,), _N, jnp.int32)
    for _ in range(12):
        mid = (lo + hi) >> 1
        v = plsc.load_gather(bt_v, [mid])
        p = v < b
        lo = jnp.where(p, mid + 1, lo)
        hi = jnp.where(p, hi, mid)
    return lo


def _row_ctx(i, xs_v, ys_v, zs_v, bt_v, bnd_v):
    iv = _splat(i)
    xi = plsc.load_gather(xs_v, [iv])
    yi = plsc.load_gather(ys_v, [iv])
    zi = plsc.load_gather(zs_v, [iv])
    bi = plsc.load_gather(bt_v, [iv])
    s = jnp.max(plsc.load_gather(bnd_v, [bi]))
    e = jnp.max(plsc.load_gather(bnd_v, [bi + 1]))
    return iv, xi, yi, zi, s, e


def _chunk_mask(base, it, iv, xi, yi, zi, s, e, xs_v, ys_v, zs_v):
    jv = base + it
    dx = xs_v[pl.ds(base, 16)] - xi
    dy = ys_v[pl.ds(base, 16)] - yi
    dz = zs_v[pl.ds(base, 16)] - zi
    d2 = dx * dx + dy * dy + dz * dz
    m = (jv >= s) & (jv < e) & (jv != iv) & (d2 < _CUT2)
    return jv, d2, m


def _sqrt16(d2):
    y = plsc.bitcast(0x5F3759DF - (plsc.bitcast(d2, jnp.int32) >> 1),
                     jnp.float32)
    for _ in range(3):
        y = y * (1.5 - 0.5 * d2 * y * y)
    return jnp.where(d2 > 0, d2 * y, 0.0)


def _count_body(xs_h, ys_h, zs_h, bt_h, tot_h, xs_v, ys_v, zs_v, bt_v, bnd_v,
                tot_v, csem):
    wid = lax.axis_index("s") * _NC + lax.axis_index("c")
    d0 = pltpu.async_copy(xs_h, xs_v.at[pl.ds(0, _N)], csem)
    d1 = pltpu.async_copy(ys_h, ys_v.at[pl.ds(0, _N)], csem)
    d2 = pltpu.async_copy(zs_h, zs_v.at[pl.ds(0, _N)], csem)
    d3 = pltpu.async_copy(bt_h, bt_v, csem)
    d0.wait()
    d1.wait()
    d2.wait()
    d3.wait()
    bnd_v[...] = _seg_bounds(bt_v)
    it = _it16()
    row0 = wid * _RPW

    def row_body(i, acc):
        iv, xi, yi, zi, s, e = _row_ctx(i, xs_v, ys_v, zs_v, bt_v, bnd_v)
        c0 = s >> 4
        nch = ((e + 15) >> 4) - c0

        def chunk(k, a):
            base = (c0 + 4 * k) * 16
            for u in range(4):
                _, _, m = _chunk_mask(base + u * 16, it, iv, xi, yi, zi, s, e,
                                      xs_v, ys_v, zs_v)
                a = a + plsc.all_reduce_population_count(m)
            return a

        return lax.fori_loop(0, (nch + 3) >> 2, chunk, acc)

    tot = lax.fori_loop(row0, row0 + _RPW, row_body, jnp.zeros((16,), jnp.int32))
    tot_v[...] = jnp.zeros((16,), jnp.int32) + jnp.max(tot)
    pltpu.sync_copy(tot_v, tot_h.at[pl.ds(wid * 16, 16)])


_params = pltpu.CompilerParams(needs_layout_passes=False)

_count_call = pl.kernel(
    _count_body,
    out_type=jax.ShapeDtypeStruct((_NW * 16,), jnp.int32),
    mesh=_mesh,
    compiler_params=_params,
    scratch_types=[
        pltpu.VMEM((_N + 128,), jnp.float32),
        pltpu.VMEM((_N + 128,), jnp.float32),
        pltpu.VMEM((_N + 128,), jnp.float32),
        pltpu.VMEM((_N,), jnp.int32),
        pltpu.VMEM((16,), jnp.int32),
        pltpu.VMEM((16,), jnp.int32),
        pltpu.SemaphoreType.DMA,
    ],
)


def _emit_body(xs_h, ys_h, zs_h, bt_h, tot_h, ei0_h, ei1_h, ew_h,
               xs_v, ys_v, zs_v, bt_v, bnd_v, tot_v,
               stji, std, vi_v, vj_v, vf_v,
               idx16, vih, vjh, vfh, cneg_v, czer_v,
               sem0, sem1, sem2):
    wid = lax.axis_index("s") * _NC + lax.axis_index("c")
    d0 = pltpu.async_copy(xs_h, xs_v.at[pl.ds(0, _N)], sem0)
    d1 = pltpu.async_copy(ys_h, ys_v.at[pl.ds(0, _N)], sem1)
    d2 = pltpu.async_copy(zs_h, zs_v.at[pl.ds(0, _N)], sem2)
    d3 = pltpu.async_copy(bt_h, bt_v, sem0)
    d4 = pltpu.async_copy(tot_h, tot_v, sem1)
    d0.wait()
    d1.wait()
    d2.wait()
    d3.wait()
    d4.wait()
    bnd_v[...] = _seg_bounds(bt_v)
    it = _it16()

    def tot_body(c, carry):
        ba, ta = carry
        ch = tot_v[pl.ds(c * 16, 16)]
        lane0 = it == 0
        v0 = jnp.where(lane0, ch, 0)
        return (ba + jnp.where(c < wid, v0, 0), ta + v0)

    ba, ta = lax.fori_loop(0, _NW, tot_body,
                           (jnp.zeros((16,), jnp.int32),
                            jnp.zeros((16,), jnp.int32)))
    base = jnp.sum(ba)
    total = jnp.minimum(jnp.sum(ta), _MAXP)

    def const_body(k, _):
        sl = pl.ds(k * 16, 16)
        cneg_v[sl] = jnp.full((16,), -1, jnp.int32)
        czer_v[sl] = jnp.zeros((16,), jnp.float32)
        return 0

    lax.fori_loop(0, 128, const_body, 0)

    def scat16(pos0, cnt, soff):
        gi = soff + it
        ji = plsc.load_gather(stji, [gi])
        vjh[...] = ji & 4095
        vih[...] = lax.shift_right_logical(ji, 12)
        vfh[...] = _sqrt16(plsc.load_gather(std, [gi]))
        idx16[...] = jnp.where(it < cnt, pos0 + it, _MAXP)
        d0 = pltpu.async_copy(vih, ei0_h.at[idx16], sem0)
        d1 = pltpu.async_copy(vjh, ei1_h.at[idx16], sem1)
        d2_ = pltpu.async_copy(vfh, ew_h.at[idx16], sem2)
        d0.wait()
        d1.wait()
        d2_.wait()

    pad_lo = jnp.maximum(wid * _SLICE, total)
    pad_hi = (wid + 1) * _SLICE
    npad = jnp.maximum(pad_hi - pad_lo, 0)
    phead = jnp.minimum((8 - pad_lo % 8) % 8, npad)

    @pl.when(phead > 0)
    def _():
        idx16[...] = jnp.where(it < phead, pad_lo + it, _MAXP)
        vfh[...] = jnp.zeros((16,), jnp.float32)
        d0 = pltpu.async_copy(cneg_v.at[pl.ds(0, 16)], ei0_h.at[idx16], sem0)
        d1 = pltpu.async_copy(cneg_v.at[pl.ds(0, 16)], ei1_h.at[idx16], sem1)
        d2_ = pltpu.async_copy(vfh, ew_h.at[idx16], sem2)
        d0.wait()
        d1.wait()
        d2_.wait()

    pal = pl.multiple_of(pad_lo + phead, 8)
    plen = jnp.maximum(pad_hi - pal, 0)

    def pad_chunk(k, _):
        cur_ = pl.multiple_of(pal + k * 2048, 8)
        d0 = pltpu.async_copy(cneg_v, ei0_h.at[pl.ds(cur_, 2048)], sem0)
        d1 = pltpu.async_copy(cneg_v, ei1_h.at[pl.ds(cur_, 2048)], sem1)
        d2 = pltpu.async_copy(czer_v, ew_h.at[pl.ds(cur_, 2048)], sem2)
        d0.wait()
        d1.wait()
        d2.wait()
        return 0

    lax.fori_loop(0, plen >> 11, pad_chunk, 0)
    pcur = pal + ((plen >> 11) << 11)
    for sz in (1024, 512, 256, 128, 64, 32, 16, 8):
        pred = (plen & sz) != 0

        @pl.when(pred)
        def _(pcur=pl.multiple_of(pcur, 8), sz=sz):
            d0 = pltpu.async_copy(cneg_v.at[pl.ds(0, sz)],
                                  ei0_h.at[pl.ds(pcur, sz)], sem0)
            d1 = pltpu.async_copy(cneg_v.at[pl.ds(0, sz)],
                                  ei1_h.at[pl.ds(pcur, sz)], sem1)
            d2 = pltpu.async_copy(czer_v.at[pl.ds(0, sz)],
                                  ew_h.at[pl.ds(pcur, sz)], sem2)
            d0.wait()
            d1.wait()
            d2.wait()

        pcur = pcur + jnp.where(pred, sz, 0)

    def flush(n_f, fb):
        a = jnp.clip(_MAXP - fb, 0, n_f)
        head = jnp.minimum((8 - fb % 8) % 8, a)

        @pl.when(head > 0)
        def _():
            scat16(fb, head, 0)

        lf = (a - head) & -8

        def copy_body(k, _):
            gi = head + k * 16 + it
            dst = pl.ds(k * 16, 16)
            ji = plsc.load_gather(stji, [gi])
            vj_v[dst] = ji & 4095
            vi_v[dst] = lax.shift_right_logical(ji, 12)
            vf_v[dst] = _sqrt16(plsc.load_gather(std, [gi]))
            return 0

        lax.fori_loop(0, (lf + 15) >> 4, copy_body, 0)

        cur = fb + head
        soff = jnp.int32(0)
        for sz in (8192, 4096, 2048, 1024, 512, 256, 128, 64, 32, 16, 8):
            pred = (lf & sz) != 0

            @pl.when(pred)
            def _(cur=pl.multiple_of(cur, 8), soff=pl.multiple_of(soff, 8),
                  sz=sz):
                d0 = pltpu.async_copy(vi_v.at[pl.ds(soff, sz)],
                                      ei0_h.at[pl.ds(cur, sz)], sem0)
                d1 = pltpu.async_copy(vj_v.at[pl.ds(soff, sz)],
                                      ei1_h.at[pl.ds(cur, sz)], sem1)
                d2 = pltpu.async_copy(vf_v.at[pl.ds(soff, sz)],
                                      ew_h.at[pl.ds(cur, sz)], sem2)
                d0.wait()
                d1.wait()
                d2.wait()

            inc = jnp.where(pred, sz, 0)
            cur = cur + inc
            soff = soff + inc

        rem = a - head - lf

        @pl.when(rem > 0)
        def _(cur=cur):
            scat16(cur, rem, head + lf)

    row0 = wid * _RPW

    def row_body(i, carry):
        wp_v, fb = carry
        iv, xi, yi, zi, s, e = _row_ctx(i, xs_v, ys_v, zs_v, bt_v, bnd_v)
        iv12 = iv * 4096
        c0 = s >> 4
        nch = ((e + 15) >> 4) - c0

        def chunk(k, wpv):
            base = (c0 + 8 * k) * 16
            jvs, d2s, ms = [], [], []
            for u in range(8):
                jv, d2, m = _chunk_mask(base + u * 16, it, iv, xi, yi, zi,
                                        s, e, xs_v, ys_v, zs_v)
                jvs.append(jv)
                d2s.append(d2)
                ms.append(m)

            css = [plsc.cumsum(jnp.where(m, 1, 0)) for m in ms]
            pcs = [plsc.all_reduce_population_count(m) for m in ms]
            for u in range(8):
                pos = wpv + css[u] - 1
                plsc.store_scatter(stji, [pos], iv12 + jvs[u], mask=ms[u])
                plsc.store_scatter(std, [pos], d2s[u], mask=ms[u])
                wpv = wpv + pcs[u]
            return wpv

        wp_v = lax.fori_loop(0, (nch + 7) >> 3, chunk, wp_v)
        wp_s = jnp.max(wp_v)

        def do_flush(args):
            wpv_, fb_ = args
            flush(_FLUSH_T, fb_)
            nshift = jnp.max(wpv_) - _FLUSH_T

            def shift_body(k, _):
                src = pl.ds(_FLUSH_T + k * 16, 16)
                dst = pl.ds(k * 16, 16)
                stji[dst] = stji[src]
                std[dst] = std[src]
                return 0

            lax.fori_loop(0, (nshift + 15) >> 4, shift_body, 0)
            return wpv_ - _FLUSH_T, fb_ + _FLUSH_T

        return lax.cond(wp_s >= _FLUSH_T, do_flush, lambda a_: a_, (wp_v, fb))

    wp_v, fb = lax.fori_loop(row0, row0 + _RPW, row_body,
                             (jnp.zeros((16,), jnp.int32), base))
    flush(jnp.max(wp_v), fb)


_emit_call = pl.kernel(
    _emit_body,
    out_type=(
        jax.ShapeDtypeStruct((_OUT,), jnp.int32),
        jax.ShapeDtypeStruct((_OUT,), jnp.int32),
        jax.ShapeDtypeStruct((_OUT,), jnp.float32),
    ),
    mesh=_mesh,
    compiler_params=_params,
    scratch_types=[
        pltpu.VMEM((_N + 128,), jnp.float32),
        pltpu.VMEM((_N + 128,), jnp.float32),
        pltpu.VMEM((_N + 128,), jnp.float32),
        pltpu.VMEM((_N,), jnp.int32),
        pltpu.VMEM((16,), jnp.int32),
        pltpu.VMEM((_NW * 16,), jnp.int32),
        pltpu.VMEM((_STAGE,), jnp.int32),
        pltpu.VMEM((_STAGE,), jnp.float32),
        pltpu.VMEM((_STAGE,), jnp.int32),
        pltpu.VMEM((_STAGE,), jnp.int32),
        pltpu.VMEM((_STAGE,), jnp.float32),
        pltpu.VMEM((16,), jnp.int32),
        pltpu.VMEM((16,), jnp.int32),
        pltpu.VMEM((16,), jnp.int32),
        pltpu.VMEM((16,), jnp.float32),
        pltpu.VMEM((2048,), jnp.int32),
        pltpu.VMEM((2048,), jnp.float32),
        pltpu.SemaphoreType.DMA,
        pltpu.SemaphoreType.DMA,
        pltpu.SemaphoreType.DMA,
    ],
)


def kernel(pos, batch):
    pt = pos.astype(jnp.float32).T
    xs, ys, zs = pt[0], pt[1], pt[2]
    bt = batch.astype(jnp.int32)
    tot = _count_call(xs, ys, zs, bt)
    ei0, ei1, ew = _emit_call(xs, ys, zs, bt, tot)
    edge_index = jnp.stack([ei0[:_MAXP], ei1[:_MAXP]])
    edge_weight = ew[:_MAXP]
    return edge_index, edge_weight, None
